# baseline (speedup 1.0000x reference)
import jax
import jax.numpy as jnp
from jax import lax
from jax.experimental import pallas as pl
from jax.experimental.pallas import tpu as pltpu

_BM1 = 2048
_BM2 = 2048
_N_BISECT = 18


def _stage1_kernel(k_ref, train_ref, xt_ref, w1_ref, b1_ref, maskt_ref,
                   be1t_ref, cmax_ref):
    bet = jnp.dot(w1_ref[...], xt_ref[...],
                  preferred_element_type=jnp.float32)
    bet = jnp.maximum(bet + b1_ref[...], 0.0)

    kf = k_ref[0].astype(jnp.float32)
    colmax = jnp.max(bet, axis=0, keepdims=True)
    lo = jnp.zeros_like(colmax)
    ones_row = jnp.ones((1, bet.shape[0]), jnp.float32)

    def body(_, carry):
        lo, hi = carry
        mid = 0.5 * (lo + hi)
        ind = (bet > mid).astype(jnp.float32)
        cnt = jnp.dot(ones_row, ind, preferred_element_type=jnp.float32)
        take = cnt >= kf
        return jnp.where(take, mid, lo), jnp.where(take, hi, mid)

    lo, _ = lax.fori_loop(0, _N_BISECT, body, (lo, colmax), unroll=2)

    keep = (bet > lo) | (maskt_ref[...] != 0) | (train_ref[0] == 0)
    be1t = jnp.where(keep, bet, 0.0)
    be1t_ref[...] = be1t.astype(jnp.bfloat16)

    bmax = jnp.max(be1t, axis=1, keepdims=True)

    @pl.when(pl.program_id(0) == 0)
    def _():
        cmax_ref[...] = bmax

    @pl.when(pl.program_id(0) != 0)
    def _():
        cmax_ref[...] = jnp.maximum(cmax_ref[...], bmax)


def _stage2_kernel(xt_ref, be1t_ref, cmax_ref, w4_ref, b4_ref,
                   norm_ref, loss_ref):
    inv = 1.0 / (cmax_ref[...] + 1e-9)
    nrmt = be1t_ref[...].astype(jnp.float32) * inv
    norm_ref[...] = nrmt.T
    projt = jnp.dot(w4_ref[...], nrmt.astype(jnp.float8_e4m3fn),
                    preferred_element_type=jnp.float32) + b4_ref[...]
    d = projt - xt_ref[...].astype(jnp.float32)
    s = jnp.sum(d * d)

    @pl.when(pl.program_id(0) == 0)
    def _():
        loss_ref[0] = s

    @pl.when(pl.program_id(0) != 0)
    def _():
        loss_ref[0] = loss_ref[0] + s


def kernel(precomputed_embeddings, text_names, mask, training, device, k,
           W1, b1, W4, b4):
    x = precomputed_embeddings
    b, d_in = x.shape
    d_lat = W1.shape[0]

    k_arr = jnp.asarray(k, jnp.int32).reshape(1)
    t_arr = jnp.asarray(training, jnp.int32).reshape(1)
    xtb = x.T.astype(jnp.bfloat16)
    w1b = W1.astype(jnp.bfloat16)
    b1c = b1.reshape(d_lat, 1)
    maskt = mask.T.astype(jnp.int8)

    be1t, cmax = pl.pallas_call(
        _stage1_kernel,
        grid=(b // _BM1,),
        in_specs=[
            pl.BlockSpec(memory_space=pltpu.SMEM),
            pl.BlockSpec(memory_space=pltpu.SMEM),
            pl.BlockSpec((d_in, _BM1), lambda i: (0, i)),
            pl.BlockSpec((d_lat, d_in), lambda i: (0, 0)),
            pl.BlockSpec((d_lat, 1), lambda i: (0, 0)),
            pl.BlockSpec((d_lat, _BM1), lambda i: (0, i)),
        ],
        out_specs=[
            pl.BlockSpec((d_lat, _BM1), lambda i: (0, i)),
            pl.BlockSpec((d_lat, 1), lambda i: (0, 0)),
        ],
        out_shape=[
            jax.ShapeDtypeStruct((d_lat, b), jnp.bfloat16),
            jax.ShapeDtypeStruct((d_lat, 1), jnp.float32),
        ],
        compiler_params=pltpu.CompilerParams(
            dimension_semantics=("arbitrary",)),
    )(k_arr, t_arr, xtb, w1b, b1c, maskt)

    w4b = W4.astype(jnp.float8_e4m3fn)
    b4c = b4.reshape(d_in, 1)

    norm, loss_sum = pl.pallas_call(
        _stage2_kernel,
        grid=(b // _BM2,),
        in_specs=[
            pl.BlockSpec((d_in, _BM2), lambda i: (0, i)),
            pl.BlockSpec((d_lat, _BM2), lambda i: (0, i)),
            pl.BlockSpec((d_lat, 1), lambda i: (0, 0)),
            pl.BlockSpec((d_in, d_lat), lambda i: (0, 0)),
            pl.BlockSpec((d_in, 1), lambda i: (0, 0)),
        ],
        out_specs=[
            pl.BlockSpec((_BM2, d_lat), lambda i: (i, 0)),
            pl.BlockSpec(memory_space=pltpu.SMEM),
        ],
        out_shape=[
            jax.ShapeDtypeStruct((b, d_lat), jnp.float32),
            jax.ShapeDtypeStruct((1,), jnp.float32),
        ],
        compiler_params=pltpu.CompilerParams(
            dimension_semantics=("arbitrary",)),
    )(xtb, be1t, cmax, w4b, b4c)

    loss = loss_sum[0] / jnp.float32(b * d_in)
    return (norm, loss, text_names)


# speedup vs baseline: 1.0034x; 1.0034x over previous
import jax
import jax.numpy as jnp
from jax import lax
from jax.experimental import pallas as pl
from jax.experimental.pallas import tpu as pltpu

_BM1 = 2048
_BM2 = 2048
_N_BISECT = 18


def _stage1_kernel(k_ref, train_ref, xt_ref, w1_ref, b1_ref, maskt_ref,
                   be1t_ref, cmax_ref):
    bet = jnp.dot(w1_ref[...], xt_ref[...],
                  preferred_element_type=jnp.float32)
    bet = jnp.maximum(bet + b1_ref[...], 0.0)

    kf = k_ref[0].astype(jnp.float32)
    colmax = jnp.max(bet, axis=0, keepdims=True)
    lo = jnp.zeros_like(colmax)
    ones_row = jnp.ones((1, bet.shape[0]), jnp.float32)

    def body(_, carry):
        lo, hi = carry
        mid = 0.5 * (lo + hi)
        ind = (bet > mid).astype(jnp.float32)
        cnt = jnp.dot(ones_row, ind, preferred_element_type=jnp.float32)
        take = cnt >= kf
        return jnp.where(take, mid, lo), jnp.where(take, hi, mid)

    lo, _ = lax.fori_loop(0, _N_BISECT, body, (lo, colmax), unroll=2)

    keep = (bet > lo) | (maskt_ref[...] != 0) | (train_ref[0] == 0)
    be1t = jnp.where(keep, bet, 0.0)
    be1t_ref[...] = be1t.astype(jnp.bfloat16)

    bmax = jnp.max(be1t, axis=1, keepdims=True)

    @pl.when(pl.program_id(0) == 0)
    def _():
        cmax_ref[...] = bmax

    @pl.when(pl.program_id(0) != 0)
    def _():
        cmax_ref[...] = jnp.maximum(cmax_ref[...], bmax)


def _stage2_kernel(xt_ref, be1t_ref, cmax_ref, w4_ref, b4_ref,
                   norm_ref, loss_ref):
    inv = 1.0 / (cmax_ref[...] + 1e-9)
    nrmt = be1t_ref[...].astype(jnp.float32) * inv
    norm_ref[...] = nrmt.T
    projt = jnp.dot(w4_ref[...], nrmt.astype(jnp.bfloat16),
                    preferred_element_type=jnp.float32) + b4_ref[...]
    d = projt - xt_ref[...].astype(jnp.float32)
    s = jnp.sum(d * d)

    @pl.when(pl.program_id(0) == 0)
    def _():
        loss_ref[0] = s

    @pl.when(pl.program_id(0) != 0)
    def _():
        loss_ref[0] = loss_ref[0] + s


def kernel(precomputed_embeddings, text_names, mask, training, device, k,
           W1, b1, W4, b4):
    x = precomputed_embeddings
    b, d_in = x.shape
    d_lat = W1.shape[0]

    k_arr = jnp.asarray(k, jnp.int32).reshape(1)
    t_arr = jnp.asarray(training, jnp.int32).reshape(1)
    xtb = x.T.astype(jnp.bfloat16)
    w1b = W1.astype(jnp.bfloat16)
    b1c = b1.reshape(d_lat, 1)
    maskt = mask.T.astype(jnp.int8)

    be1t, cmax = pl.pallas_call(
        _stage1_kernel,
        grid=(b // _BM1,),
        in_specs=[
            pl.BlockSpec(memory_space=pltpu.SMEM),
            pl.BlockSpec(memory_space=pltpu.SMEM),
            pl.BlockSpec((d_in, _BM1), lambda i: (0, i)),
            pl.BlockSpec((d_lat, d_in), lambda i: (0, 0)),
            pl.BlockSpec((d_lat, 1), lambda i: (0, 0)),
            pl.BlockSpec((d_lat, _BM1), lambda i: (0, i)),
        ],
        out_specs=[
            pl.BlockSpec((d_lat, _BM1), lambda i: (0, i)),
            pl.BlockSpec((d_lat, 1), lambda i: (0, 0)),
        ],
        out_shape=[
            jax.ShapeDtypeStruct((d_lat, b), jnp.bfloat16),
            jax.ShapeDtypeStruct((d_lat, 1), jnp.float32),
        ],
        compiler_params=pltpu.CompilerParams(
            dimension_semantics=("arbitrary",)),
    )(k_arr, t_arr, xtb, w1b, b1c, maskt)

    w4b = W4.astype(jnp.bfloat16)
    b4c = b4.reshape(d_in, 1)

    norm, loss_sum = pl.pallas_call(
        _stage2_kernel,
        grid=(b // _BM2,),
        in_specs=[
            pl.BlockSpec((d_in, _BM2), lambda i: (0, i)),
            pl.BlockSpec((d_lat, _BM2), lambda i: (0, i)),
            pl.BlockSpec((d_lat, 1), lambda i: (0, 0)),
            pl.BlockSpec((d_in, d_lat), lambda i: (0, 0)),
            pl.BlockSpec((d_in, 1), lambda i: (0, 0)),
        ],
        out_specs=[
            pl.BlockSpec((_BM2, d_lat), lambda i: (i, 0)),
            pl.BlockSpec(memory_space=pltpu.SMEM),
        ],
        out_shape=[
            jax.ShapeDtypeStruct((b, d_lat), jnp.float32),
            jax.ShapeDtypeStruct((1,), jnp.float32),
        ],
        compiler_params=pltpu.CompilerParams(
            dimension_semantics=("arbitrary",)),
    )(xtb, be1t, cmax, w4b, b4c)

    loss = loss_sum[0] / jnp.float32(b * d_in)
    return (norm, loss, text_names)
